# Initial kernel scaffold; baseline (speedup 1.0000x reference)
import jax
import jax.numpy as jnp
from jax.experimental import pallas as pl


def kernel(x, edge_index):
    raise NotImplementedError("write your pallas kernel here")


# speedup vs baseline: 9.6796x; 9.6796x over previous
import functools

import jax
import jax.numpy as jnp
from jax import lax
from jax.experimental import pallas as pl
from jax.experimental.pallas import tpu as pltpu
from jax.experimental.pallas import tpu_sc as plsc

N = 10000
DEG = 32
D = 128
S = 8
EPS = 1e-06

NC, NS, L = 2, 16, 16
NW = NC * NS
RPW = -(-N // NW)
CPW = RPW * DEG
OPW = RPW * S
NPAD = 10240
ROWBLK = 1024


def _logw_body(x_ref, o_ref):
    xb = x_ref[...]
    o_ref[...] = jnp.log(jnp.sum(xb * xb, axis=1, keepdims=True) + EPS)


def _log_weights(x_pad):
    return pl.pallas_call(
        _logw_body,
        out_shape=jax.ShapeDtypeStruct((NPAD, 1), jnp.float32),
        grid=(NPAD // ROWBLK,),
        in_specs=[pl.BlockSpec((ROWBLK, D), lambda g: (g, 0))],
        out_specs=pl.BlockSpec((ROWBLK, 1), lambda g: (g, 0)),
    )(x_pad)


def _sc_body(logw_hbm, col_hbm, gum_hbm, out_hbm, logw_v, col_v, gum_v, dst_v):
    wid = lax.axis_index("s") * NC + lax.axis_index("c")
    base = wid * CPW
    pltpu.sync_copy(logw_hbm, logw_v)
    pltpu.sync_copy(col_hbm.at[pl.ds(base, CPW)], col_v)
    pltpu.sync_copy(gum_hbm.at[pl.ds(base, CPW)], gum_v)
    lanes = lax.iota(jnp.int32, L)
    m8 = lanes < S

    def row(r, carry):
        off = r * DEG
        iA = plsc.load_gather(col_v, [off + lanes])
        iB = plsc.load_gather(col_v, [off + L + lanes])
        iA = jnp.minimum(jnp.maximum(iA, 0), N - 1)
        iB = jnp.minimum(jnp.maximum(iB, 0), N - 1)
        kA = plsc.load_gather(logw_v, [iA]) + plsc.load_gather(gum_v, [off + lanes])
        kB = plsc.load_gather(logw_v, [iB]) + plsc.load_gather(gum_v, [off + L + lanes])
        sA, wA = plsc.sort_key_val(kA, iA, descending=True)
        sB, wB = plsc.sort_key_val(kB, iB, descending=False)
        take = sA >= sB
        kM = jnp.where(take, sA, sB)
        vM = jnp.where(take, wA, wB)
        _, top = plsc.sort_key_val(kM, vM, descending=True)
        plsc.store_scatter(dst_v, [r * S + lanes], top, mask=m8)
        return carry

    lax.fori_loop(0, RPW, row, 0)
    pltpu.sync_copy(dst_v, out_hbm.at[pl.ds(wid * OPW, OPW)])


def _sc_sample(logw, colp, gump):
    mesh = plsc.VectorSubcoreMesh(core_axis_name="c", subcore_axis_name="s")
    k = functools.partial(
        pl.kernel,
        out_type=jax.ShapeDtypeStruct((NW * OPW,), jnp.int32),
        mesh=mesh,
        compiler_params=pltpu.CompilerParams(needs_layout_passes=False),
        scratch_types=[
            pltpu.VMEM((NPAD,), jnp.float32),
            pltpu.VMEM((CPW,), jnp.int32),
            pltpu.VMEM((CPW,), jnp.float32),
            pltpu.VMEM((OPW,), jnp.int32),
        ],
    )(_sc_body)
    return k(logw, colp, gump)


def kernel(x, edge_index):
    col = edge_index[1]
    x_pad = jnp.concatenate(
        [x.astype(jnp.float32), jnp.zeros((NPAD - N, D), jnp.float32)], axis=0
    )
    logw = _log_weights(x_pad).reshape(NPAD)

    u = jax.random.uniform(jax.random.key(42), (N, DEG), minval=1e-20, maxval=1.0)
    gumbel = (-jnp.log(-jnp.log(u))).reshape(-1)

    pad = NW * CPW - N * DEG
    colp = jnp.concatenate([col.astype(jnp.int32), jnp.zeros((pad,), jnp.int32)])
    gump = jnp.concatenate([gumbel, jnp.zeros((pad,), jnp.float32)])

    dst = _sc_sample(logw, colp, gump)[: N * S]
    src = jnp.repeat(jnp.arange(N, dtype=jnp.int32), S)
    return jnp.stack([src, dst])


# speedup vs baseline: 11.6470x; 1.2033x over previous
import functools

import jax
import jax.numpy as jnp
from jax import lax
from jax.experimental import pallas as pl
from jax.experimental.pallas import tpu as pltpu
from jax.experimental.pallas import tpu_sc as plsc

N = 10000
DEG = 32
D = 128
S = 8
EPS = 1e-06

NC, NS, L = 2, 16, 16
NW = NC * NS
R_BIG, R_SML = 320, 304
NBIG = 17
QUADS_SML = R_SML // 4
GWIN = R_BIG // 4 + 8
GROWS = 2560

_KS0 = 0
_KS1 = 42
_KS2 = 0x1BD11BDA ^ _KS0 ^ _KS1
_ROTS = ((13, 15, 26, 6), (17, 29, 16, 24))


def _shr(x, n):
    return lax.shift_right_logical(x, jnp.full(x.shape, n, jnp.int32))


def _rotl(x, n):
    return jnp.left_shift(x, n) | _shr(x, 32 - n)


def _threefry_bits(cnt):
    ks = (jnp.int32(_KS0), jnp.int32(_KS1), jnp.int32(_KS2))
    x0 = jnp.zeros_like(cnt) + ks[0]
    x1 = cnt + ks[1]
    for rnd in range(5):
        for r in _ROTS[rnd % 2]:
            x0 = x0 + x1
            x1 = _rotl(x1, r) ^ x0
        x0 = x0 + ks[(rnd + 1) % 3]
        x1 = x1 + ks[(rnd + 2) % 3] + jnp.int32(rnd + 1)
    return x0 ^ x1


def _logw_body(x_ref, o_ref):
    xb = x_ref[...]
    o_ref[...] = jnp.log(jnp.sum(xb * xb, axis=1, keepdims=True) + EPS)


def _log_weights(x):
    return pl.pallas_call(
        _logw_body,
        out_shape=jax.ShapeDtypeStruct((N, 1), jnp.float32),
        grid=(25,),
        in_specs=[pl.BlockSpec((N // 25, D), lambda g: (g, 0))],
        out_specs=pl.BlockSpec((N // 25, 1), lambda g: (g, 0)),
    )(x)


def _gum_body(o_ref):
    g = pl.program_id(0)
    blk = GROWS // 10
    r = lax.broadcasted_iota(jnp.int32, (blk, D), 0)
    c = lax.broadcasted_iota(jnp.int32, (blk, D), 1)
    cnt = (g * blk + r) * D + c
    bits = _threefry_bits(cnt)
    fl = _shr(bits, 9) | jnp.full(bits.shape, 0x3F800000, jnp.int32)
    uf = lax.bitcast_convert_type(fl, jnp.float32) - jnp.float32(1.0)
    mn = jnp.float32(1e-20)
    u = jnp.maximum(mn, uf * (jnp.float32(1.0) - mn) + mn)
    o_ref[...] = -jnp.log(-jnp.log(u))


def _gumbel_table():
    return pl.pallas_call(
        _gum_body,
        out_shape=jax.ShapeDtypeStruct((GROWS, D), jnp.float32),
        grid=(10,),
        out_specs=pl.BlockSpec((GROWS // 10, D), lambda g: (g, 0)),
    )()


def _sc_body(logw_hbm, col_hbm, gum_hbm, dst_hbm, src_hbm,
             logw_v, col_v, gum_v, dst_v, src_v):
    w = lax.axis_index("s") * NC + lax.axis_index("c")
    big = w < NBIG
    base = R_SML * w + (R_BIG - R_SML) * jnp.minimum(w, NBIG)
    lanes = lax.iota(jnp.int32, L)
    m8 = lanes < S
    gstart = pl.multiple_of((base // 32) * 8, 8)
    gdelta = base // 4 - gstart

    pltpu.sync_copy(logw_hbm, logw_v)
    pltpu.sync_copy(gum_hbm.at[pl.ds(gstart, GWIN)], gum_v)

    @pl.when(big)
    def _():
        pltpu.sync_copy(col_hbm.at[pl.ds(base * DEG, R_BIG * DEG)],
                        col_v.at[pl.ds(0, R_BIG * DEG)])

    @pl.when(jnp.logical_not(big))
    def _():
        pltpu.sync_copy(col_hbm.at[pl.ds(base * DEG, R_SML * DEG)],
                        col_v.at[pl.ds(0, R_SML * DEG)])

    def do_row(r):
        off = r * DEG
        p = off + lanes
        q = p + L
        iA = plsc.load_gather(col_v, [p])
        iB = plsc.load_gather(col_v, [q])
        gA = plsc.load_gather(gum_v, [gdelta + _shr(p, 7), p & 127])
        gB = plsc.load_gather(gum_v, [gdelta + _shr(q, 7), q & 127])
        kA = plsc.load_gather(logw_v, [iA]) + gA
        kB = plsc.load_gather(logw_v, [iB]) + gB
        sA, vA = plsc.sort_key_val(kA, iA, descending=True)
        sB, vB = plsc.sort_key_val(kB, iB)
        take = sA >= sB
        kM = jnp.where(take, sA, sB)
        vM = jnp.where(take, vA, vB)
        _, top = plsc.sort_key_val(kM, vM, descending=True)
        o = r * S + lanes
        plsc.store_scatter(dst_v, [o], top, mask=m8)
        plsc.store_scatter(src_v, [o], jnp.zeros((L,), jnp.int32) + (base + r),
                           mask=m8)

    def quad(qi, carry):
        for j in range(4):
            do_row(qi * 4 + j)
        return carry

    lax.fori_loop(0, QUADS_SML, quad, 0)

    @pl.when(big)
    def _():
        lax.fori_loop(QUADS_SML, R_BIG // 4, quad, 0)
        pltpu.sync_copy(dst_v.at[pl.ds(0, R_BIG * S)],
                        dst_hbm.at[pl.ds(base * S, R_BIG * S)])
        pltpu.sync_copy(src_v.at[pl.ds(0, R_BIG * S)],
                        src_hbm.at[pl.ds(base * S, R_BIG * S)])

    @pl.when(jnp.logical_not(big))
    def _():
        pltpu.sync_copy(dst_v.at[pl.ds(0, R_SML * S)],
                        dst_hbm.at[pl.ds(base * S, R_SML * S)])
        pltpu.sync_copy(src_v.at[pl.ds(0, R_SML * S)],
                        src_hbm.at[pl.ds(base * S, R_SML * S)])


def _sc_sample(logw, col, gum):
    mesh = plsc.VectorSubcoreMesh(core_axis_name="c", subcore_axis_name="s")
    k = functools.partial(
        pl.kernel,
        out_type=(
            jax.ShapeDtypeStruct((N * S,), jnp.int32),
            jax.ShapeDtypeStruct((N * S,), jnp.int32),
        ),
        mesh=mesh,
        compiler_params=pltpu.CompilerParams(needs_layout_passes=False),
        scratch_types=[
            pltpu.VMEM((N,), jnp.float32),
            pltpu.VMEM((R_BIG * DEG,), jnp.int32),
            pltpu.VMEM((GWIN, D), jnp.float32),
            pltpu.VMEM((R_BIG * S,), jnp.int32),
            pltpu.VMEM((R_BIG * S,), jnp.int32),
        ],
    )(_sc_body)
    return k(logw, col, gum)


def kernel(x, edge_index):
    col = edge_index[1]
    logw = _log_weights(x).reshape(N)
    gum = _gumbel_table()
    dst, src = _sc_sample(logw, col, gum)
    return jnp.stack([src, dst])


# speedup vs baseline: 15.7498x; 1.3523x over previous
import functools

import jax
import jax.numpy as jnp
from jax import lax
from jax.experimental import pallas as pl
from jax.experimental.pallas import tpu as pltpu
from jax.experimental.pallas import tpu_sc as plsc

N = 10000
DEG = 32
D = 128
S = 8
EPS = 1e-06

NC, NS, L = 2, 16, 16
NW = NC * NS
R_BIG, R_SML = 320, 304
NBIG = 17
QUADS_SML = R_SML // 4
GWIN = R_BIG // 4 + 8
GROWS = 2560

_KS0 = 0
_KS1 = 42
_KS2 = 0x1BD11BDA ^ _KS0 ^ _KS1
_ROTS = ((13, 15, 26, 6), (17, 29, 16, 24))


def _shr(x, n):
    return lax.shift_right_logical(x, jnp.full(x.shape, n, jnp.int32))


def _rotl(x, n):
    return jnp.left_shift(x, n) | _shr(x, 32 - n)


def _threefry_bits(cnt):
    ks = (jnp.int32(_KS0), jnp.int32(_KS1), jnp.int32(_KS2))
    x0 = jnp.zeros_like(cnt) + ks[0]
    x1 = cnt + ks[1]
    for rnd in range(5):
        for r in _ROTS[rnd % 2]:
            x0 = x0 + x1
            x1 = _rotl(x1, r) ^ x0
        x0 = x0 + ks[(rnd + 1) % 3]
        x1 = x1 + ks[(rnd + 2) % 3] + jnp.int32(rnd + 1)
    return x0 ^ x1


def _col_body(e_ref, o_ref):
    o_ref[...] = e_ref[1]


def _col_extract(edge_index):
    return pl.pallas_call(
        _col_body,
        out_shape=jax.ShapeDtypeStruct((N * DEG,), jnp.int32),
    )(edge_index)


def _logw_body(x_ref, o_ref):
    xb = x_ref[...]
    o_ref[...] = jnp.log(jnp.sum(xb * xb, axis=1, keepdims=True) + EPS)


def _log_weights(x):
    return pl.pallas_call(
        _logw_body,
        out_shape=jax.ShapeDtypeStruct((N, 1), jnp.float32),
        grid=(5,),
        in_specs=[pl.BlockSpec((N // 5, D), lambda g: (g, 0))],
        out_specs=pl.BlockSpec((N // 5, 1), lambda g: (g, 0)),
    )(x)


def _gum_body(o_ref):
    g = pl.program_id(0)
    blk = GROWS // 10
    r = lax.broadcasted_iota(jnp.int32, (blk, D), 0)
    c = lax.broadcasted_iota(jnp.int32, (blk, D), 1)
    cnt = (g * blk + r) * D + c
    bits = _threefry_bits(cnt)
    fl = _shr(bits, 9) | jnp.full(bits.shape, 0x3F800000, jnp.int32)
    uf = lax.bitcast_convert_type(fl, jnp.float32) - jnp.float32(1.0)
    mn = jnp.float32(1e-20)
    u = jnp.maximum(mn, uf * (jnp.float32(1.0) - mn) + mn)
    o_ref[...] = -jnp.log(-jnp.log(u))


def _gumbel_table():
    return pl.pallas_call(
        _gum_body,
        out_shape=jax.ShapeDtypeStruct((GROWS, D), jnp.float32),
        grid=(10,),
        out_specs=pl.BlockSpec((GROWS // 10, D), lambda g: (g, 0)),
    )()


def _sc_body(logw_hbm, col_hbm, gum_hbm, dst_hbm, src_hbm,
             logw_v, col_v, gum_v, dst_v, src_v):
    w = lax.axis_index("s") * NC + lax.axis_index("c")
    big = w < NBIG
    base = R_SML * w + (R_BIG - R_SML) * jnp.minimum(w, NBIG)
    lanes = lax.iota(jnp.int32, L)
    m8 = lanes < S
    gstart = pl.multiple_of((base // 32) * 8, 8)
    gdelta = base // 4 - gstart

    pltpu.sync_copy(logw_hbm, logw_v)
    pltpu.sync_copy(gum_hbm.at[pl.ds(gstart, GWIN)], gum_v)

    @pl.when(big)
    def _():
        pltpu.sync_copy(col_hbm.at[pl.ds(base * DEG, R_BIG * DEG)],
                        col_v.at[pl.ds(0, R_BIG * DEG)])

    @pl.when(jnp.logical_not(big))
    def _():
        pltpu.sync_copy(col_hbm.at[pl.ds(base * DEG, R_SML * DEG)],
                        col_v.at[pl.ds(0, R_SML * DEG)])

    def do_row(r):
        off = r * DEG
        p = off + lanes
        q = p + L
        iA = plsc.load_gather(col_v, [p])
        iB = plsc.load_gather(col_v, [q])
        gA = plsc.load_gather(gum_v, [gdelta + _shr(p, 7), p & 127])
        gB = plsc.load_gather(gum_v, [gdelta + _shr(q, 7), q & 127])
        kA = plsc.load_gather(logw_v, [iA]) + gA
        kB = plsc.load_gather(logw_v, [iB]) + gB
        sA, vA = plsc.sort_key_val(kA, iA, descending=True)
        sB, vB = plsc.sort_key_val(kB, iB)
        take = sA >= sB
        kM = jnp.where(take, sA, sB)
        vM = jnp.where(take, vA, vB)
        _, top = plsc.sort_key_val(kM, vM, descending=True)
        o = r * S + lanes
        plsc.store_scatter(dst_v, [o], top, mask=m8)
        plsc.store_scatter(src_v, [o], jnp.zeros((L,), jnp.int32) + (base + r),
                           mask=m8)

    def quad(qi, carry):
        for j in range(4):
            do_row(qi * 4 + j)
        return carry

    lax.fori_loop(0, QUADS_SML, quad, 0)

    @pl.when(big)
    def _():
        lax.fori_loop(QUADS_SML, R_BIG // 4, quad, 0)
        pltpu.sync_copy(dst_v.at[pl.ds(0, R_BIG * S)],
                        dst_hbm.at[pl.ds(base * S, R_BIG * S)])
        pltpu.sync_copy(src_v.at[pl.ds(0, R_BIG * S)],
                        src_hbm.at[pl.ds(base * S, R_BIG * S)])

    @pl.when(jnp.logical_not(big))
    def _():
        pltpu.sync_copy(dst_v.at[pl.ds(0, R_SML * S)],
                        dst_hbm.at[pl.ds(base * S, R_SML * S)])
        pltpu.sync_copy(src_v.at[pl.ds(0, R_SML * S)],
                        src_hbm.at[pl.ds(base * S, R_SML * S)])


def _sc_sample(logw, col, gum):
    mesh = plsc.VectorSubcoreMesh(core_axis_name="c", subcore_axis_name="s")
    k = functools.partial(
        pl.kernel,
        out_type=(
            jax.ShapeDtypeStruct((N * S,), jnp.int32),
            jax.ShapeDtypeStruct((N * S,), jnp.int32),
        ),
        mesh=mesh,
        compiler_params=pltpu.CompilerParams(needs_layout_passes=False),
        scratch_types=[
            pltpu.VMEM((N,), jnp.float32),
            pltpu.VMEM((R_BIG * DEG,), jnp.int32),
            pltpu.VMEM((GWIN, D), jnp.float32),
            pltpu.VMEM((R_BIG * S,), jnp.int32),
            pltpu.VMEM((R_BIG * S,), jnp.int32),
        ],
    )(_sc_body)
    return k(logw, col, gum)


def kernel(x, edge_index):
    col = _col_extract(edge_index)
    logw = _log_weights(x).reshape(N)
    gum = _gumbel_table()
    dst, src = _sc_sample(logw, col, gum)
    return jnp.stack([src, dst])


# speedup vs baseline: 17.3026x; 1.0986x over previous
import functools

import jax
import jax.numpy as jnp
from jax import lax
from jax.experimental import pallas as pl
from jax.experimental.pallas import tpu as pltpu
from jax.experimental.pallas import tpu_sc as plsc

N = 10000
DEG = 32
D = 128
S = 8
EPS = 1e-06

NC, NS, L = 2, 16, 16
NW = NC * NS
R_BIG, R_SML = 320, 304
NBIG = 17
QUADS_SML = R_SML // 4
GWIN = R_BIG // 4 + 8
GROWS = 2560

_KS0 = 0
_KS1 = 42
_KS2 = 0x1BD11BDA ^ _KS0 ^ _KS1
_ROTS = ((13, 15, 26, 6), (17, 29, 16, 24))


def _shr(x, n):
    return lax.shift_right_logical(x, jnp.full(x.shape, n, jnp.int32))


def _rotl(x, n):
    return jnp.left_shift(x, n) | _shr(x, 32 - n)


def _threefry_bits(cnt):
    ks = (jnp.int32(_KS0), jnp.int32(_KS1), jnp.int32(_KS2))
    x0 = jnp.zeros_like(cnt) + ks[0]
    x1 = cnt + ks[1]
    for rnd in range(5):
        for r in _ROTS[rnd % 2]:
            x0 = x0 + x1
            x1 = _rotl(x1, r) ^ x0
        x0 = x0 + ks[(rnd + 1) % 3]
        x1 = x1 + ks[(rnd + 2) % 3] + jnp.int32(rnd + 1)
    return x0 ^ x1


def _col_body(e_ref, o_ref):
    o_ref[...] = e_ref[1]


def _col_extract(edge_index):
    return pl.pallas_call(
        _col_body,
        out_shape=jax.ShapeDtypeStruct((N * DEG,), jnp.int32),
    )(edge_index)


def _logw_body(x_ref, o_ref):
    xb = x_ref[...]
    lw = jnp.log(jnp.sum(xb * xb, axis=1, keepdims=True) + EPS)
    o_ref[...] = jnp.reshape(lw, (N,))


def _log_weights(x):
    return pl.pallas_call(
        _logw_body,
        out_shape=jax.ShapeDtypeStruct((N,), jnp.float32),
    )(x)


def _gum_body(o_ref):
    g = pl.program_id(0)
    blk = GROWS // 10
    r = lax.broadcasted_iota(jnp.int32, (blk, D), 0)
    c = lax.broadcasted_iota(jnp.int32, (blk, D), 1)
    cnt = (g * blk + r) * D + c
    bits = _threefry_bits(cnt)
    fl = _shr(bits, 9) | jnp.full(bits.shape, 0x3F800000, jnp.int32)
    uf = lax.bitcast_convert_type(fl, jnp.float32) - jnp.float32(1.0)
    mn = jnp.float32(1e-20)
    u = jnp.maximum(mn, uf * (jnp.float32(1.0) - mn) + mn)
    o_ref[...] = -jnp.log(-jnp.log(u))


def _gumbel_table():
    return pl.pallas_call(
        _gum_body,
        out_shape=jax.ShapeDtypeStruct((GROWS, D), jnp.float32),
        grid=(10,),
        out_specs=pl.BlockSpec((GROWS // 10, D), lambda g: (g, 0)),
    )()


def _sc_body(logw_hbm, col_hbm, gum_hbm, dst_hbm, src_hbm,
             logw_v, col_v, gum_v, dst_v, src_v):
    w = lax.axis_index("s") * NC + lax.axis_index("c")
    big = w < NBIG
    base = R_SML * w + (R_BIG - R_SML) * jnp.minimum(w, NBIG)
    lanes = lax.iota(jnp.int32, L)
    m8 = lanes < S
    gstart = pl.multiple_of((base // 32) * 8, 8)
    gdelta = base // 4 - gstart

    pltpu.sync_copy(logw_hbm, logw_v)
    pltpu.sync_copy(gum_hbm.at[pl.ds(gstart, GWIN)], gum_v)

    @pl.when(big)
    def _():
        pltpu.sync_copy(col_hbm.at[pl.ds(base * DEG, R_BIG * DEG)],
                        col_v.at[pl.ds(0, R_BIG * DEG)])

    @pl.when(jnp.logical_not(big))
    def _():
        pltpu.sync_copy(col_hbm.at[pl.ds(base * DEG, R_SML * DEG)],
                        col_v.at[pl.ds(0, R_SML * DEG)])

    def do_row(r):
        off = r * DEG
        p = off + lanes
        q = p + L
        iA = plsc.load_gather(col_v, [p])
        iB = plsc.load_gather(col_v, [q])
        gA = plsc.load_gather(gum_v, [gdelta + _shr(p, 7), p & 127])
        gB = plsc.load_gather(gum_v, [gdelta + _shr(q, 7), q & 127])
        kA = plsc.load_gather(logw_v, [iA]) + gA
        kB = plsc.load_gather(logw_v, [iB]) + gB
        sA, vA = plsc.sort_key_val(kA, iA, descending=True)
        sB, vB = plsc.sort_key_val(kB, iB)
        take = sA >= sB
        kM = jnp.where(take, sA, sB)
        vM = jnp.where(take, vA, vB)
        _, top = plsc.sort_key_val(kM, vM, descending=True)
        o = r * S + lanes
        plsc.store_scatter(dst_v, [o], top, mask=m8)
        plsc.store_scatter(src_v, [o], jnp.zeros((L,), jnp.int32) + (base + r),
                           mask=m8)

    def quad(qi, carry):
        for j in range(4):
            do_row(qi * 4 + j)
        return carry

    lax.fori_loop(0, QUADS_SML, quad, 0)

    @pl.when(big)
    def _():
        lax.fori_loop(QUADS_SML, R_BIG // 4, quad, 0)
        pltpu.sync_copy(dst_v.at[pl.ds(0, R_BIG * S)],
                        dst_hbm.at[pl.ds(base * S, R_BIG * S)])
        pltpu.sync_copy(src_v.at[pl.ds(0, R_BIG * S)],
                        src_hbm.at[pl.ds(base * S, R_BIG * S)])

    @pl.when(jnp.logical_not(big))
    def _():
        pltpu.sync_copy(dst_v.at[pl.ds(0, R_SML * S)],
                        dst_hbm.at[pl.ds(base * S, R_SML * S)])
        pltpu.sync_copy(src_v.at[pl.ds(0, R_SML * S)],
                        src_hbm.at[pl.ds(base * S, R_SML * S)])


def _sc_sample(logw, col, gum):
    mesh = plsc.VectorSubcoreMesh(core_axis_name="c", subcore_axis_name="s")
    k = functools.partial(
        pl.kernel,
        out_type=(
            jax.ShapeDtypeStruct((N * S,), jnp.int32),
            jax.ShapeDtypeStruct((N * S,), jnp.int32),
        ),
        mesh=mesh,
        compiler_params=pltpu.CompilerParams(needs_layout_passes=False),
        scratch_types=[
            pltpu.VMEM((N,), jnp.float32),
            pltpu.VMEM((R_BIG * DEG,), jnp.int32),
            pltpu.VMEM((GWIN, D), jnp.float32),
            pltpu.VMEM((R_BIG * S,), jnp.int32),
            pltpu.VMEM((R_BIG * S,), jnp.int32),
        ],
    )(_sc_body)
    return k(logw, col, gum)


def kernel(x, edge_index):
    col = _col_extract(edge_index)
    logw = _log_weights(x)
    gum = _gumbel_table()
    dst, src = _sc_sample(logw, col, gum)
    return jnp.stack([src, dst])


# speedup vs baseline: 17.3248x; 1.0013x over previous
import functools

import jax
import jax.numpy as jnp
from jax import lax
from jax.experimental import pallas as pl
from jax.experimental.pallas import tpu as pltpu
from jax.experimental.pallas import tpu_sc as plsc

N = 10000
DEG = 32
D = 128
S = 8
EPS = 1e-06

NC, NS, L = 2, 16, 16
NW = NC * NS
R_BIG, R_SML = 320, 304
NBIG = 17
QUADS_SML = R_SML // 4
GWIN = R_BIG // 4 + 8
GROWS = 2560

_KS0 = 0
_KS1 = 42
_KS2 = 0x1BD11BDA ^ _KS0 ^ _KS1
_ROTS = ((13, 15, 26, 6), (17, 29, 16, 24))


def _shr(x, n):
    return lax.shift_right_logical(x, jnp.full(x.shape, n, jnp.int32))


def _rotl(x, n):
    return jnp.left_shift(x, n) | _shr(x, 32 - n)


def _threefry_bits(cnt):
    ks = (jnp.int32(_KS0), jnp.int32(_KS1), jnp.int32(_KS2))
    x0 = jnp.zeros_like(cnt) + ks[0]
    x1 = cnt + ks[1]
    for rnd in range(5):
        for r in _ROTS[rnd % 2]:
            x0 = x0 + x1
            x1 = _rotl(x1, r) ^ x0
        x0 = x0 + ks[(rnd + 1) % 3]
        x1 = x1 + ks[(rnd + 2) % 3] + jnp.int32(rnd + 1)
    return x0 ^ x1


def _col_body(e_ref, o_ref):
    o_ref[...] = e_ref[1]


def _col_extract(edge_index):
    return pl.pallas_call(
        _col_body,
        out_shape=jax.ShapeDtypeStruct((N * DEG,), jnp.int32),
    )(edge_index)


def _logw_body(x_ref, o_ref):
    xb = x_ref[...]
    lw = jnp.log(jnp.sum(xb * xb, axis=1, keepdims=True) + EPS)
    o_ref[...] = jnp.reshape(lw, (N,))


def _log_weights(x):
    return pl.pallas_call(
        _logw_body,
        out_shape=jax.ShapeDtypeStruct((N,), jnp.float32),
    )(x)


def _gum_body(o_ref):
    g = pl.program_id(0)
    blk = GROWS // 10
    r = lax.broadcasted_iota(jnp.int32, (blk, D), 0)
    c = lax.broadcasted_iota(jnp.int32, (blk, D), 1)
    cnt = (g * blk + r) * D + c
    bits = _threefry_bits(cnt)
    fl = _shr(bits, 9) | jnp.full(bits.shape, 0x3F800000, jnp.int32)
    uf = lax.bitcast_convert_type(fl, jnp.float32) - jnp.float32(1.0)
    mn = jnp.float32(1e-20)
    u = jnp.maximum(mn, uf * (jnp.float32(1.0) - mn) + mn)
    o_ref[...] = -jnp.log(-jnp.log(u))


def _gumbel_table():
    return pl.pallas_call(
        _gum_body,
        out_shape=jax.ShapeDtypeStruct((GROWS, D), jnp.float32),
        grid=(10,),
        out_specs=pl.BlockSpec((GROWS // 10, D), lambda g: (g, 0)),
    )()


def _sc_body(logw_hbm, col_hbm, gum_hbm, dst_hbm, src_hbm,
             logw_v, col_v, gum_v, dst_v, src_v):
    w = lax.axis_index("s") * NC + lax.axis_index("c")
    big = w < NBIG
    base = R_SML * w + (R_BIG - R_SML) * jnp.minimum(w, NBIG)
    lanes = lax.iota(jnp.int32, L)
    m8 = lanes < S
    gstart = pl.multiple_of((base // 32) * 8, 8)
    gdelta = base // 4 - gstart

    pltpu.sync_copy(logw_hbm, logw_v)
    pltpu.sync_copy(gum_hbm.at[pl.ds(gstart, GWIN)], gum_v)

    @pl.when(big)
    def _():
        pltpu.sync_copy(col_hbm.at[pl.ds(base * DEG, R_BIG * DEG)],
                        col_v.at[pl.ds(0, R_BIG * DEG)])

    @pl.when(jnp.logical_not(big))
    def _():
        pltpu.sync_copy(col_hbm.at[pl.ds(base * DEG, R_SML * DEG)],
                        col_v.at[pl.ds(0, R_SML * DEG)])

    def do_row(r):
        off = r * DEG
        grow = gdelta + off // 128
        gcol = off % 128
        iA = col_v[pl.ds(off, L)]
        iB = col_v[pl.ds(off + L, L)]
        gA = gum_v[grow, pl.ds(gcol, L)]
        gB = gum_v[grow, pl.ds(gcol + L, L)]
        kA = plsc.load_gather(logw_v, [iA]) + gA
        kB = plsc.load_gather(logw_v, [iB]) + gB
        sA, vA = plsc.sort_key_val(kA, iA, descending=True)
        sB, vB = plsc.sort_key_val(kB, iB)
        take = sA >= sB
        kM = jnp.where(take, sA, sB)
        vM = jnp.where(take, vA, vB)
        _, top = plsc.sort_key_val(kM, vM, descending=True)
        o = r * S + lanes
        plsc.store_scatter(dst_v, [o], top, mask=m8)
        plsc.store_scatter(src_v, [o], jnp.zeros((L,), jnp.int32) + (base + r),
                           mask=m8)

    UNROLL = 8
    def oct_(qi, carry):
        for j in range(UNROLL):
            do_row(qi * UNROLL + j)
        return carry

    lax.fori_loop(0, R_SML // UNROLL, oct_, 0)

    @pl.when(big)
    def _():
        lax.fori_loop(R_SML // UNROLL, R_BIG // UNROLL, oct_, 0)
        pltpu.sync_copy(dst_v.at[pl.ds(0, R_BIG * S)],
                        dst_hbm.at[pl.ds(base * S, R_BIG * S)])
        pltpu.sync_copy(src_v.at[pl.ds(0, R_BIG * S)],
                        src_hbm.at[pl.ds(base * S, R_BIG * S)])

    @pl.when(jnp.logical_not(big))
    def _():
        pltpu.sync_copy(dst_v.at[pl.ds(0, R_SML * S)],
                        dst_hbm.at[pl.ds(base * S, R_SML * S)])
        pltpu.sync_copy(src_v.at[pl.ds(0, R_SML * S)],
                        src_hbm.at[pl.ds(base * S, R_SML * S)])


def _sc_sample(logw, col, gum):
    mesh = plsc.VectorSubcoreMesh(core_axis_name="c", subcore_axis_name="s")
    k = functools.partial(
        pl.kernel,
        out_type=(
            jax.ShapeDtypeStruct((N * S,), jnp.int32),
            jax.ShapeDtypeStruct((N * S,), jnp.int32),
        ),
        mesh=mesh,
        compiler_params=pltpu.CompilerParams(needs_layout_passes=False),
        scratch_types=[
            pltpu.VMEM((N,), jnp.float32),
            pltpu.VMEM((R_BIG * DEG,), jnp.int32),
            pltpu.VMEM((GWIN, D), jnp.float32),
            pltpu.VMEM((R_BIG * S,), jnp.int32),
            pltpu.VMEM((R_BIG * S,), jnp.int32),
        ],
    )(_sc_body)
    return k(logw, col, gum)


def kernel(x, edge_index):
    col = _col_extract(edge_index)
    logw = _log_weights(x)
    gum = _gumbel_table()
    dst, src = _sc_sample(logw, col, gum)
    return jnp.stack([src, dst])


# speedup vs baseline: 20.3790x; 1.1763x over previous
import functools

import jax
import jax.numpy as jnp
from jax import lax
from jax.experimental import pallas as pl
from jax.experimental.pallas import tpu as pltpu
from jax.experimental.pallas import tpu_sc as plsc

N = 10000
DEG = 32
D = 128
S = 8
EPS = 1e-06

NC, NS, L = 2, 16, 16
NW = NC * NS
R_BIG, R_SML = 320, 304
NBIG = 17
QUADS_SML = R_SML // 4
GWIN = R_BIG // 4 + 8
GROWS = 2560

_KS0 = 0
_KS1 = 42
_KS2 = 0x1BD11BDA ^ _KS0 ^ _KS1
_ROTS = ((13, 15, 26, 6), (17, 29, 16, 24))


def _shr(x, n):
    return lax.shift_right_logical(x, jnp.full(x.shape, n, jnp.int32))


def _rotl(x, n):
    return jnp.left_shift(x, n) | _shr(x, 32 - n)


def _threefry_bits(cnt):
    ks = (jnp.int32(_KS0), jnp.int32(_KS1), jnp.int32(_KS2))
    x0 = jnp.zeros_like(cnt) + ks[0]
    x1 = cnt + ks[1]
    for rnd in range(5):
        for r in _ROTS[rnd % 2]:
            x0 = x0 + x1
            x1 = _rotl(x1, r) ^ x0
        x0 = x0 + ks[(rnd + 1) % 3]
        x1 = x1 + ks[(rnd + 2) % 3] + jnp.int32(rnd + 1)
    return x0 ^ x1


def _col_body(e_ref, o_ref):
    o_ref[...] = e_ref[1]


def _col_extract(edge_index):
    return pl.pallas_call(
        _col_body,
        out_shape=jax.ShapeDtypeStruct((N * DEG,), jnp.int32),
    )(edge_index)


def _logw_body(x_ref, o_ref):
    xb = x_ref[...]
    lw = jnp.log(jnp.sum(xb * xb, axis=1, keepdims=True) + EPS)
    o_ref[...] = jnp.reshape(lw, (N,))


def _log_weights(x):
    return pl.pallas_call(
        _logw_body,
        out_shape=jax.ShapeDtypeStruct((N,), jnp.float32),
    )(x)


def _gum_body(o_ref):
    g = pl.program_id(0)
    blk = GROWS // 10
    r = lax.broadcasted_iota(jnp.int32, (blk, D), 0)
    c = lax.broadcasted_iota(jnp.int32, (blk, D), 1)
    cnt = (g * blk + r) * D + c
    bits = _threefry_bits(cnt)
    fl = _shr(bits, 9) | jnp.full(bits.shape, 0x3F800000, jnp.int32)
    uf = lax.bitcast_convert_type(fl, jnp.float32) - jnp.float32(1.0)
    mn = jnp.float32(1e-20)
    u = jnp.maximum(mn, uf * (jnp.float32(1.0) - mn) + mn)
    o_ref[...] = -jnp.log(-jnp.log(u))


def _gumbel_table():
    return pl.pallas_call(
        _gum_body,
        out_shape=jax.ShapeDtypeStruct((GROWS, D), jnp.float32),
        grid=(10,),
        out_specs=pl.BlockSpec((GROWS // 10, D), lambda g: (g, 0)),
    )()


def _sc_body(logw_hbm, col_hbm, gum_hbm, dst_hbm, src_hbm,
             logw_v, col_v, gum_v, dst_v, src_v):
    w = lax.axis_index("s") * NC + lax.axis_index("c")
    big = w < NBIG
    base = R_SML * w + (R_BIG - R_SML) * jnp.minimum(w, NBIG)
    lanes = lax.iota(jnp.int32, L)
    m8 = lanes < S
    gstart = pl.multiple_of((base // 32) * 8, 8)
    gdelta = base // 4 - gstart

    pltpu.sync_copy(logw_hbm, logw_v)
    pltpu.sync_copy(gum_hbm.at[pl.ds(gstart, GWIN)], gum_v)

    @pl.when(big)
    def _():
        pltpu.sync_copy(col_hbm.at[pl.ds(base * DEG, R_BIG * DEG)],
                        col_v.at[pl.ds(0, R_BIG * DEG)])

    @pl.when(jnp.logical_not(big))
    def _():
        pltpu.sync_copy(col_hbm.at[pl.ds(base * DEG, R_SML * DEG)],
                        col_v.at[pl.ds(0, R_SML * DEG)])

    def do_row(r):
        off = r * DEG
        grow = gdelta + off // 128
        gcol = off % 128
        iA = col_v[pl.ds(off, L)]
        iB = col_v[pl.ds(off + L, L)]
        gA = gum_v[grow, pl.ds(gcol, L)]
        gB = gum_v[grow, pl.ds(gcol + L, L)]
        kA = plsc.load_gather(logw_v, [iA]) + gA
        kB = plsc.load_gather(logw_v, [iB]) + gB
        sA, vA = plsc.sort_key_val(kA, iA, descending=True)
        sB, vB = plsc.sort_key_val(kB, iB)
        take = sA >= sB
        kM = jnp.where(take, sA, sB)
        vM = jnp.where(take, vA, vB)
        _, top = plsc.sort_key_val(kM, vM, descending=True)
        o = r * S + lanes
        plsc.store_scatter(dst_v, [o], top, mask=m8)
        plsc.store_scatter(src_v, [o], jnp.zeros((L,), jnp.int32) + (base + r),
                           mask=m8)

    @plsc.parallel_loop(0, R_SML, 1, unroll=8)
    def _(r):
        do_row(r)

    @pl.when(big)
    def _():
        @plsc.parallel_loop(R_SML, R_BIG, 1, unroll=8)
        def _(r):
            do_row(r)
        pltpu.sync_copy(dst_v.at[pl.ds(0, R_BIG * S)],
                        dst_hbm.at[pl.ds(base * S, R_BIG * S)])
        pltpu.sync_copy(src_v.at[pl.ds(0, R_BIG * S)],
                        src_hbm.at[pl.ds(base * S, R_BIG * S)])

    @pl.when(jnp.logical_not(big))
    def _():
        pltpu.sync_copy(dst_v.at[pl.ds(0, R_SML * S)],
                        dst_hbm.at[pl.ds(base * S, R_SML * S)])
        pltpu.sync_copy(src_v.at[pl.ds(0, R_SML * S)],
                        src_hbm.at[pl.ds(base * S, R_SML * S)])


def _sc_sample(logw, col, gum):
    mesh = plsc.VectorSubcoreMesh(core_axis_name="c", subcore_axis_name="s")
    k = functools.partial(
        pl.kernel,
        out_type=(
            jax.ShapeDtypeStruct((N * S,), jnp.int32),
            jax.ShapeDtypeStruct((N * S,), jnp.int32),
        ),
        mesh=mesh,
        compiler_params=pltpu.CompilerParams(needs_layout_passes=False),
        scratch_types=[
            pltpu.VMEM((N,), jnp.float32),
            pltpu.VMEM((R_BIG * DEG,), jnp.int32),
            pltpu.VMEM((GWIN, D), jnp.float32),
            pltpu.VMEM((R_BIG * S,), jnp.int32),
            pltpu.VMEM((R_BIG * S,), jnp.int32),
        ],
    )(_sc_body)
    return k(logw, col, gum)


def kernel(x, edge_index):
    col = _col_extract(edge_index)
    logw = _log_weights(x)
    gum = _gumbel_table()
    dst, src = _sc_sample(logw, col, gum)
    return jnp.stack([src, dst])


# speedup vs baseline: 20.5832x; 1.0100x over previous
import functools

import jax
import jax.numpy as jnp
from jax import lax
from jax.experimental import pallas as pl
from jax.experimental.pallas import tpu as pltpu
from jax.experimental.pallas import tpu_sc as plsc

N = 10000
DEG = 32
D = 128
S = 8
EPS = 1e-06

NC, NS, L = 2, 16, 16
NW = NC * NS
R_BIG, R_SML = 320, 304
NBIG = 17
QUADS_SML = R_SML // 4
GWIN = R_BIG // 4 + 8
GROWS = 2560

_KS0 = 0
_KS1 = 42
_KS2 = 0x1BD11BDA ^ _KS0 ^ _KS1
_ROTS = ((13, 15, 26, 6), (17, 29, 16, 24))


def _shr(x, n):
    return lax.shift_right_logical(x, jnp.full(x.shape, n, jnp.int32))


def _rotl(x, n):
    return jnp.left_shift(x, n) | _shr(x, 32 - n)


def _threefry_bits(cnt):
    ks = (jnp.int32(_KS0), jnp.int32(_KS1), jnp.int32(_KS2))
    x0 = jnp.zeros_like(cnt) + ks[0]
    x1 = cnt + ks[1]
    for rnd in range(5):
        for r in _ROTS[rnd % 2]:
            x0 = x0 + x1
            x1 = _rotl(x1, r) ^ x0
        x0 = x0 + ks[(rnd + 1) % 3]
        x1 = x1 + ks[(rnd + 2) % 3] + jnp.int32(rnd + 1)
    return x0 ^ x1


def _col_body(e_ref, o_ref):
    o_ref[...] = e_ref[1]


def _col_extract(edge_index):
    return pl.pallas_call(
        _col_body,
        out_shape=jax.ShapeDtypeStruct((N * DEG,), jnp.int32),
    )(edge_index)


LWBLK = 2048
NPAD = 10240


def _logw_body(x_ref, o_ref):
    xb = x_ref[...]
    lw = jnp.log(jnp.sum(xb * xb, axis=1, keepdims=True) + EPS)
    o_ref[...] = jnp.reshape(lw, (LWBLK,))


def _log_weights(x):
    return pl.pallas_call(
        _logw_body,
        out_shape=jax.ShapeDtypeStruct((NPAD,), jnp.float32),
        grid=(NPAD // LWBLK,),
        in_specs=[pl.BlockSpec((LWBLK, D), lambda g: (g, 0))],
        out_specs=pl.BlockSpec((LWBLK,), lambda g: (g,)),
    )(x)


def _gum_body(o_ref):
    g = pl.program_id(0)
    blk = GROWS // 4
    r = lax.broadcasted_iota(jnp.int32, (blk, D), 0)
    c = lax.broadcasted_iota(jnp.int32, (blk, D), 1)
    cnt = (g * blk + r) * D + c
    bits = _threefry_bits(cnt)
    fl = _shr(bits, 9) | jnp.full(bits.shape, 0x3F800000, jnp.int32)
    uf = lax.bitcast_convert_type(fl, jnp.float32) - jnp.float32(1.0)
    mn = jnp.float32(1e-20)
    u = jnp.maximum(mn, uf * (jnp.float32(1.0) - mn) + mn)
    o_ref[...] = -jnp.log(-jnp.log(u))


def _gumbel_table():
    return pl.pallas_call(
        _gum_body,
        out_shape=jax.ShapeDtypeStruct((GROWS, D), jnp.float32),
        grid=(4,),
        out_specs=pl.BlockSpec((GROWS // 4, D), lambda g: (g, 0)),
    )()


def _sc_body(logw_hbm, col_hbm, gum_hbm, dst_hbm, src_hbm,
             logw_v, col_v, gum_v, dst_v, src_v):
    w = lax.axis_index("s") * NC + lax.axis_index("c")
    big = w < NBIG
    base = R_SML * w + (R_BIG - R_SML) * jnp.minimum(w, NBIG)
    lanes = lax.iota(jnp.int32, L)
    m8 = lanes < S
    gstart = pl.multiple_of((base // 32) * 8, 8)
    gdelta = base // 4 - gstart

    pltpu.sync_copy(logw_hbm, logw_v)
    pltpu.sync_copy(gum_hbm.at[pl.ds(gstart, GWIN)], gum_v)

    @pl.when(big)
    def _():
        pltpu.sync_copy(col_hbm.at[pl.ds(base * DEG, R_BIG * DEG)],
                        col_v.at[pl.ds(0, R_BIG * DEG)])

    @pl.when(jnp.logical_not(big))
    def _():
        pltpu.sync_copy(col_hbm.at[pl.ds(base * DEG, R_SML * DEG)],
                        col_v.at[pl.ds(0, R_SML * DEG)])

    def do_row(r):
        off = r * DEG
        grow = gdelta + off // 128
        gcol = off % 128
        iA = col_v[pl.ds(off, L)]
        iB = col_v[pl.ds(off + L, L)]
        gA = gum_v[grow, pl.ds(gcol, L)]
        gB = gum_v[grow, pl.ds(gcol + L, L)]
        kA = plsc.load_gather(logw_v, [iA]) + gA
        kB = plsc.load_gather(logw_v, [iB]) + gB
        sA, vA = plsc.sort_key_val(kA, iA, descending=True)
        sB, vB = plsc.sort_key_val(kB, iB)
        take = sA >= sB
        kM = jnp.where(take, sA, sB)
        vM = jnp.where(take, vA, vB)
        _, top = plsc.sort_key_val(kM, vM, descending=True)
        o = r * S + lanes
        plsc.store_scatter(dst_v, [o], top, mask=m8)
        plsc.store_scatter(src_v, [o], jnp.zeros((L,), jnp.int32) + (base + r),
                           mask=m8)

    @plsc.parallel_loop(0, R_SML, 1, unroll=8)
    def _(r):
        do_row(r)

    @pl.when(big)
    def _():
        @plsc.parallel_loop(R_SML, R_BIG, 1, unroll=8)
        def _(r):
            do_row(r)
        pltpu.sync_copy(dst_v.at[pl.ds(0, R_BIG * S)],
                        dst_hbm.at[pl.ds(base * S, R_BIG * S)])
        pltpu.sync_copy(src_v.at[pl.ds(0, R_BIG * S)],
                        src_hbm.at[pl.ds(base * S, R_BIG * S)])

    @pl.when(jnp.logical_not(big))
    def _():
        pltpu.sync_copy(dst_v.at[pl.ds(0, R_SML * S)],
                        dst_hbm.at[pl.ds(base * S, R_SML * S)])
        pltpu.sync_copy(src_v.at[pl.ds(0, R_SML * S)],
                        src_hbm.at[pl.ds(base * S, R_SML * S)])


def _sc_sample(logw, col, gum):
    mesh = plsc.VectorSubcoreMesh(core_axis_name="c", subcore_axis_name="s")
    k = functools.partial(
        pl.kernel,
        out_type=(
            jax.ShapeDtypeStruct((N * S,), jnp.int32),
            jax.ShapeDtypeStruct((N * S,), jnp.int32),
        ),
        mesh=mesh,
        compiler_params=pltpu.CompilerParams(needs_layout_passes=False),
        scratch_types=[
            pltpu.VMEM((NPAD,), jnp.float32),
            pltpu.VMEM((R_BIG * DEG,), jnp.int32),
            pltpu.VMEM((GWIN, D), jnp.float32),
            pltpu.VMEM((R_BIG * S,), jnp.int32),
            pltpu.VMEM((R_BIG * S,), jnp.int32),
        ],
    )(_sc_body)
    return k(logw, col, gum)


def kernel(x, edge_index):
    col = _col_extract(edge_index)
    logw = _log_weights(x)
    gum = _gumbel_table()
    dst, src = _sc_sample(logw, col, gum)
    return jnp.stack([src, dst])
